# Initial kernel scaffold; baseline (speedup 1.0000x reference)
import jax
import jax.numpy as jnp
from jax.experimental import pallas as pl


def kernel(x, edge_index, W1, b1, gamma1, beta1, W2, b2):
    raise NotImplementedError("write your pallas kernel here")


# speedup vs baseline: 1.0854x; 1.0854x over previous
import jax
import jax.numpy as jnp
from jax.experimental import pallas as pl


def _scale_add_kernel(a_ref, b_ref, d_ref, o_ref):
    o_ref[...] = d_ref[...] * (a_ref[...] + b_ref[...])


def kernel(x, edge_index, W1, b1, gamma1, beta1, W2, b2):
    n = x.shape[0]
    src, dst = edge_index[0], edge_index[1]
    mask = (src != dst).astype(jnp.float32)
    loop = jnp.arange(n, dtype=src.dtype)
    src = jnp.concatenate([src, loop])
    dst = jnp.concatenate([dst, loop])
    maskf = jnp.concatenate([mask, jnp.ones(n, dtype=jnp.float32)])
    deg = jnp.zeros(n, dtype=jnp.float32).at[dst].add(maskf)
    dinv = jnp.where(deg > 0, deg ** -0.5, 0.0)
    w = dinv[src] * dinv[dst] * maskf

    def spmm(h):
        return jnp.zeros((n, h.shape[1]), dtype=h.dtype).at[src].add(w[:, None] * h[dst])

    h = x @ W1 + b1
    h = spmm(h)
    mean = h.mean(axis=0)
    var = h.var(axis=0)
    h = gamma1 * (h - mean) / jnp.sqrt(var + 1e-5) + beta1
    h = jax.nn.relu(h)
    h = h @ W2 + b2
    acc = jnp.zeros((n, h.shape[1]), dtype=h.dtype).at[src[:-n]].add(w[:-n, None] * h[dst[:-n]])
    hs = (w[-n:, None]) * h
    ones = jnp.ones((n, 1), dtype=h.dtype)
    out = pl.pallas_call(
        _scale_add_kernel,
        out_shape=jax.ShapeDtypeStruct((n, h.shape[1]), h.dtype),
    )(acc, hs, ones)
    return out


# speedup vs baseline: 12.8595x; 11.8482x over previous
import functools

import jax
import jax.numpy as jnp
from jax import lax
from jax.experimental import pallas as pl
from jax.experimental.pallas import tpu as pltpu
from jax.experimental.pallas import tpu_sc as plsc

NC = 2
NS = 16
NW = NC * NS
CHK = 128


def _edges_body(ei_ref, src_ref, dst_ref, *, n, nchunks):
    src = ei_ref[0]
    dst = ei_ref[1]
    src_ref[:nchunks] = src
    dst_ref[:nchunks] = jnp.where(src == dst, n, dst)
    pad = src_ref.shape[0] - nchunks
    if pad:
        fill = jnp.full((pad, CHK), n, jnp.int32)
        src_ref[nchunks:] = fill
        dst_ref[nchunks:] = fill


def _dinv_from(degw, npad):
    deg = degw[0][:, :1] + degw[1][:, :1] + 1.0
    return lax.rsqrt(deg)


def _m1_body(x_ref, w_ref, b_ref, degw_ref, o_ref, *, n, npad):
    dinv = _dinv_from(degw_ref, npad)
    h = jnp.dot(x_ref[...], w_ref[...], preferred_element_type=jnp.float32)
    h = dinv * (h + b_ref[...])
    rowid = lax.broadcasted_iota(jnp.int32, (npad, 1), 0)
    o_ref[...] = jnp.where(rowid < n, h, 0.0)


def _l2_body(acc_ref, h1s_ref, degw_ref, g_ref, bt_ref, w_ref, b_ref, o_ref,
             *, n, npad):
    dinv = _dinv_from(degw_ref, npad)
    g = dinv * (acc_ref[0] + acc_ref[1] + h1s_ref[...])
    rowid = lax.broadcasted_iota(jnp.int32, (npad, 1), 0)
    rmask = (rowid < n).astype(jnp.float32)
    mean = jnp.sum(g, axis=0, keepdims=True) / n
    dev = (g - mean) * rmask
    var = jnp.sum(dev * dev, axis=0, keepdims=True) / n
    bn = g_ref[...] * (g - mean) * lax.rsqrt(var + 1e-5) + bt_ref[...]
    r = jnp.maximum(bn, 0.0)
    h2 = jnp.dot(r, w_ref[...], preferred_element_type=jnp.float32) + b_ref[...]
    o_ref[...] = jnp.where(rowid < n, dinv * h2, 0.0)


def _out_body(acc_ref, h2s_ref, degw_ref, o_ref, *, n, npad):
    dinv = _dinv_from(degw_ref, npad)
    o_ref[...] = (dinv * (acc_ref[0] + acc_ref[1] + h2s_ref[...]))[:n]


def _sc_deg_body(dstp_ref, zeros_ref, ones_ref, out_ref,
                 deg_sh, idx_v, ones_v, *, chpt, rpt):
    c = lax.axis_index("c")
    s = lax.axis_index("s")
    wid = s * NC + c
    rows = pl.ds(s * rpt, rpt)
    pltpu.sync_copy(zeros_ref.at[rows], deg_sh.at[rows])
    pltpu.sync_copy(ones_ref, ones_v)
    pltpu.sync_copy(dstp_ref.at[wid], idx_v)
    plsc.subcore_barrier()

    def body(j, carry):
        pltpu.sync_copy(ones_v, deg_sh.at[idx_v.at[j]], add=True)
        return carry

    lax.fori_loop(0, chpt, body, 0)
    plsc.subcore_barrier()
    pltpu.sync_copy(deg_sh.at[rows], out_ref.at[c].at[rows])


def _sc_spmm_body(srcp_ref, dstp_ref, hs_ref, zeros_ref, out_ref,
                  acc_sh, sidx_v, didx_v, rows_v, sem, *, chpt, rpt):
    c = lax.axis_index("c")
    s = lax.axis_index("s")
    wid = s * NC + c
    rows = pl.ds(s * rpt, rpt)
    pltpu.sync_copy(zeros_ref.at[rows], acc_sh.at[rows])
    pltpu.sync_copy(srcp_ref.at[wid], sidx_v)
    pltpu.sync_copy(dstp_ref.at[wid], didx_v)
    plsc.subcore_barrier()

    def body(j, carry):
        pltpu.async_copy(hs_ref.at[didx_v.at[j]], rows_v, sem).wait()
        pltpu.sync_copy(rows_v, acc_sh.at[sidx_v.at[j]], add=True)
        return carry

    lax.fori_loop(0, chpt, body, 0)
    plsc.subcore_barrier()
    pltpu.sync_copy(acc_sh.at[rows], out_ref.at[c].at[rows])


def kernel(x, edge_index, W1, b1, gamma1, beta1, W2, b2):
    n, d = x.shape
    e = edge_index.shape[1]
    nchunks = e // CHK
    assert nchunks * CHK == e
    chpt = -(-nchunks // NW)
    npad = ((n + 1 + 127) // 128) * 128
    rpt = npad // NS

    mesh = plsc.VectorSubcoreMesh(core_axis_name="c", subcore_axis_name="s")

    ei3 = edge_index.reshape(2, nchunks, CHK)
    srcp, dstp = pl.pallas_call(
        functools.partial(_edges_body, n=n, nchunks=nchunks),
        out_shape=[jax.ShapeDtypeStruct((NW * chpt, CHK), jnp.int32)] * 2,
    )(ei3)
    srcp = srcp.reshape(NW, chpt, CHK)
    dstp = dstp.reshape(NW, chpt, CHK)

    zeros16 = jnp.zeros((npad, 16), jnp.float32)
    zeros128 = jnp.zeros((npad, d), jnp.float32)
    ones16 = jnp.ones((CHK, 16), jnp.float32)
    x_pad = jnp.pad(x, ((0, npad - n), (0, 0)))

    deg_w = pl.kernel(
        functools.partial(_sc_deg_body, chpt=chpt, rpt=rpt),
        out_type=jax.ShapeDtypeStruct((NC, npad, 16), jnp.float32),
        mesh=mesh,
        compiler_params=pltpu.CompilerParams(use_tc_tiling_on_sc=False),
        scratch_types=[
            pltpu.VMEM_SHARED((npad, 16), jnp.float32),
            pltpu.VMEM((chpt, CHK), jnp.int32),
            pltpu.VMEM((CHK, 16), jnp.float32),
        ],
    )(dstp, zeros16, ones16)

    h1s = pl.pallas_call(
        functools.partial(_m1_body, n=n, npad=npad),
        out_shape=jax.ShapeDtypeStruct((npad, d), jnp.float32),
    )(x_pad, W1, b1.reshape(1, d), deg_w)

    spmm = pl.kernel(
        functools.partial(_sc_spmm_body, chpt=chpt, rpt=rpt),
        out_type=jax.ShapeDtypeStruct((NC, npad, d), jnp.float32),
        mesh=mesh,
        scratch_types=[
            pltpu.VMEM_SHARED((npad, d), jnp.float32),
            pltpu.VMEM((chpt, CHK), jnp.int32),
            pltpu.VMEM((chpt, CHK), jnp.int32),
            pltpu.VMEM((CHK, d), jnp.float32),
            pltpu.SemaphoreType.DMA,
        ],
    )

    acc1 = spmm(srcp, dstp, h1s, zeros128)

    h2s = pl.pallas_call(
        functools.partial(_l2_body, n=n, npad=npad),
        out_shape=jax.ShapeDtypeStruct((npad, d), jnp.float32),
    )(acc1, h1s, deg_w, gamma1.reshape(1, d), beta1.reshape(1, d),
      W2, b2.reshape(1, d))

    acc2 = spmm(srcp, dstp, h2s, zeros128)

    out = pl.pallas_call(
        functools.partial(_out_body, n=n, npad=npad),
        out_shape=jax.ShapeDtypeStruct((n, d), jnp.float32),
    )(acc2, h2s, deg_w)
    return out


# speedup vs baseline: 28.0724x; 2.1830x over previous
import functools

import jax
import jax.numpy as jnp
from jax import lax
from jax.experimental import pallas as pl
from jax.experimental.pallas import tpu as pltpu
from jax.experimental.pallas import tpu_sc as plsc

NC = 2
NS = 16
NW = NC * NS
CHK = 128


def _edges_body(ei_ref, src_ref, dst_ref, *, n, npad, nchunks):
    src = ei_ref[0]
    dst = ei_ref[1]
    src_ref[:nchunks] = src
    dst_ref[:nchunks] = jnp.where(src == dst, n, dst)
    pad = src_ref.shape[0] - nchunks
    if pad:
        spare = npad - n - 1
        r = lax.broadcasted_iota(jnp.int32, (pad, CHK), 0)
        c = lax.broadcasted_iota(jnp.int32, (pad, CHK), 1)
        fill = n + 1 + lax.rem(r * CHK + c, jnp.int32(spare))
        src_ref[nchunks:] = fill
        dst_ref[nchunks:] = fill


def _dinv_from(degw, npad):
    deg = degw[0][:, :1] + degw[1][:, :1] + 1.0
    return lax.rsqrt(deg)


def _m1_body(x_ref, w_ref, b_ref, degw_ref, o_ref, *, n, npad):
    dinv = _dinv_from(degw_ref, npad)
    h = jnp.dot(x_ref[...], w_ref[...], preferred_element_type=jnp.float32)
    h = dinv * (h + b_ref[...])
    rowid = lax.broadcasted_iota(jnp.int32, (npad, 1), 0)
    o_ref[...] = jnp.where(rowid < n, h, 0.0)


def _l2_body(acc_ref, h1s_ref, degw_ref, g_ref, bt_ref, w_ref, b_ref, o_ref,
             *, n, npad):
    dinv = _dinv_from(degw_ref, npad)
    g = dinv * (acc_ref[0] + acc_ref[1] + h1s_ref[...])
    rowid = lax.broadcasted_iota(jnp.int32, (npad, 1), 0)
    rmask = (rowid < n).astype(jnp.float32)
    mean = jnp.sum(g, axis=0, keepdims=True) / n
    dev = (g - mean) * rmask
    var = jnp.sum(dev * dev, axis=0, keepdims=True) / n
    bn = g_ref[...] * (g - mean) * lax.rsqrt(var + 1e-5) + bt_ref[...]
    r = jnp.maximum(bn, 0.0)
    h2 = jnp.dot(r, w_ref[...], preferred_element_type=jnp.float32) + b_ref[...]
    o_ref[...] = jnp.where(rowid < n, dinv * h2, 0.0)


def _out_body(acc_ref, h2s_ref, degw_ref, o_ref, *, n, npad):
    dinv = _dinv_from(degw_ref, npad)
    o_ref[...] = (dinv * (acc_ref[0] + acc_ref[1] + h2s_ref[...]))[:n]


def _sc_deg_body(dstp_ref, zeros_ref, ones_ref, out_ref,
                 deg_sh, idx_v, ones_v, *, chpt, rpt):
    c = lax.axis_index("c")
    s = lax.axis_index("s")
    wid = s * NC + c
    rows = pl.ds(s * rpt, rpt)
    pltpu.sync_copy(zeros_ref.at[rows], deg_sh.at[rows])
    pltpu.sync_copy(ones_ref, ones_v)
    pltpu.sync_copy(dstp_ref.at[wid], idx_v)
    plsc.subcore_barrier()

    def body(j, carry):
        pltpu.sync_copy(ones_v, deg_sh.at[idx_v.at[j]], add=True)
        return carry

    lax.fori_loop(0, chpt, body, 0)
    plsc.subcore_barrier()
    pltpu.sync_copy(deg_sh.at[rows], out_ref.at[c].at[rows])


BLK = 16


def _sc_spmm_body(srcp_ref, dstp_ref, hs_ref, zeros_ref, out_ref,
                  acc_sh, sblk, dblk, rows_v, sem0, sem1, semi, *, chpt, rpt):
    c = lax.axis_index("c")
    s = lax.axis_index("s")
    wid = s * NC + c
    nblk = chpt // BLK
    rows = pl.ds(s * rpt, rpt)
    pltpu.sync_copy(zeros_ref.at[rows], acc_sh.at[rows])
    pltpu.sync_copy(srcp_ref.at[wid].at[pl.ds(0, BLK)], sblk.at[0])
    pltpu.sync_copy(dstp_ref.at[wid].at[pl.ds(0, BLK)], dblk.at[0])
    plsc.subcore_barrier()

    def outer(k, carry):
        kb = lax.rem(k, 2)
        pltpu.async_copy(hs_ref.at[dblk.at[kb, 0]], rows_v.at[0], sem0)

        @pl.when(k + 1 < nblk)
        def _():
            nxt = pl.ds((k + 1) * BLK, BLK)
            pltpu.async_copy(srcp_ref.at[wid].at[nxt], sblk.at[1 - kb], semi)
            pltpu.async_copy(dstp_ref.at[wid].at[nxt], dblk.at[1 - kb], semi)

        def inner(i, c2):
            j = 2 * i
            pltpu.make_async_copy(
                hs_ref.at[dblk.at[kb, j]], rows_v.at[0], sem0).wait()
            pltpu.async_copy(hs_ref.at[dblk.at[kb, j + 1]], rows_v.at[1], sem1)
            pltpu.sync_copy(rows_v.at[0], acc_sh.at[sblk.at[kb, j]], add=True)
            pltpu.make_async_copy(
                hs_ref.at[dblk.at[kb, j + 1]], rows_v.at[1], sem1).wait()

            @pl.when(j + 2 < BLK)
            def _():
                pltpu.async_copy(
                    hs_ref.at[dblk.at[kb, j + 2]], rows_v.at[0], sem0)

            pltpu.sync_copy(rows_v.at[1], acc_sh.at[sblk.at[kb, j + 1]],
                            add=True)
            return c2

        lax.fori_loop(0, BLK // 2, inner, 0)

        @pl.when(k + 1 < nblk)
        def _():
            pltpu.make_async_copy(
                srcp_ref.at[wid].at[pl.ds(0, BLK)], sblk.at[1 - kb],
                semi).wait()
            pltpu.make_async_copy(
                dstp_ref.at[wid].at[pl.ds(0, BLK)], dblk.at[1 - kb],
                semi).wait()

        return carry

    lax.fori_loop(0, nblk, outer, 0)
    plsc.subcore_barrier()
    pltpu.sync_copy(acc_sh.at[rows], out_ref.at[c].at[rows])


def kernel(x, edge_index, W1, b1, gamma1, beta1, W2, b2):
    n, d = x.shape
    e = edge_index.shape[1]
    nchunks = e // CHK
    assert nchunks * CHK == e
    chpt = -(-nchunks // NW)
    chpt = ((chpt + BLK - 1) // BLK) * BLK
    npad = ((n + 1 + 127) // 128) * 128
    rpt = npad // NS

    mesh = plsc.VectorSubcoreMesh(core_axis_name="c", subcore_axis_name="s")

    ei3 = edge_index.reshape(2, nchunks, CHK)
    srcp, dstp = pl.pallas_call(
        functools.partial(_edges_body, n=n, npad=npad, nchunks=nchunks),
        out_shape=[jax.ShapeDtypeStruct((NW * chpt, CHK), jnp.int32)] * 2,
    )(ei3)
    srcp = srcp.reshape(NW, chpt, CHK)
    dstp = dstp.reshape(NW, chpt, CHK)

    zeros16 = jnp.zeros((npad, 16), jnp.float32)
    zeros128 = jnp.zeros((npad, d), jnp.float32)
    ones16 = jnp.ones((CHK, 16), jnp.float32)
    x_pad = jnp.pad(x, ((0, npad - n), (0, 0)))

    deg_w = pl.kernel(
        functools.partial(_sc_deg_body, chpt=chpt, rpt=rpt),
        out_type=jax.ShapeDtypeStruct((NC, npad, 16), jnp.float32),
        mesh=mesh,
        compiler_params=pltpu.CompilerParams(use_tc_tiling_on_sc=False),
        scratch_types=[
            pltpu.VMEM_SHARED((npad, 16), jnp.float32),
            pltpu.VMEM((chpt, CHK), jnp.int32),
            pltpu.VMEM((CHK, 16), jnp.float32),
        ],
    )(dstp, zeros16, ones16)

    h1s = pl.pallas_call(
        functools.partial(_m1_body, n=n, npad=npad),
        out_shape=jax.ShapeDtypeStruct((npad, d), jnp.float32),
    )(x_pad, W1, b1.reshape(1, d), deg_w)

    spmm = pl.kernel(
        functools.partial(_sc_spmm_body, chpt=chpt, rpt=rpt),
        out_type=jax.ShapeDtypeStruct((NC, npad, d), jnp.float32),
        mesh=mesh,
        scratch_types=[
            pltpu.VMEM_SHARED((npad, d), jnp.float32),
            pltpu.VMEM((2, BLK, CHK), jnp.int32),
            pltpu.VMEM((2, BLK, CHK), jnp.int32),
            pltpu.VMEM((2, CHK, d), jnp.float32),
            pltpu.SemaphoreType.DMA,
            pltpu.SemaphoreType.DMA,
            pltpu.SemaphoreType.DMA,
        ],
    )

    acc1 = spmm(srcp, dstp, h1s, zeros128)

    h2s = pl.pallas_call(
        functools.partial(_l2_body, n=n, npad=npad),
        out_shape=jax.ShapeDtypeStruct((npad, d), jnp.float32),
    )(acc1, h1s, deg_w, gamma1.reshape(1, d), beta1.reshape(1, d),
      W2, b2.reshape(1, d))

    acc2 = spmm(srcp, dstp, h2s, zeros128)

    out = pl.pallas_call(
        functools.partial(_out_body, n=n, npad=npad),
        out_shape=jax.ShapeDtypeStruct((n, d), jnp.float32),
    )(acc2, h2s, deg_w)
    return out


# speedup vs baseline: 30.7844x; 1.0966x over previous
import functools

import jax
import jax.numpy as jnp
from jax import lax
from jax.experimental import pallas as pl
from jax.experimental.pallas import tpu as pltpu
from jax.experimental.pallas import tpu_sc as plsc

NC = 2
NS = 16
NW = NC * NS
CHK = 128


def _edges_body(ei_ref, src_ref, dst_ref, *, n, npad, nchunks):
    src = ei_ref[0]
    dst = ei_ref[1]
    src_ref[:nchunks] = src
    dst_ref[:nchunks] = jnp.where(src == dst, n, dst)
    pad = src_ref.shape[0] - nchunks
    if pad:
        spare = npad - n - 1
        r = lax.broadcasted_iota(jnp.int32, (pad, CHK), 0)
        c = lax.broadcasted_iota(jnp.int32, (pad, CHK), 1)
        fill = n + 1 + lax.rem(r * CHK + c, jnp.int32(spare))
        src_ref[nchunks:] = fill
        dst_ref[nchunks:] = fill


def _dinv_from(degw, npad):
    deg = degw[0][:, :1] + degw[1][:, :1] + 1.0
    return lax.rsqrt(deg)


def _m1_body(x_ref, w_ref, b_ref, degw_ref, o_ref, *, n, npad):
    dinv = _dinv_from(degw_ref, npad)
    h = jnp.dot(x_ref[...], w_ref[...], preferred_element_type=jnp.float32)
    h = dinv * (h + b_ref[...])
    rowid = lax.broadcasted_iota(jnp.int32, (npad, 1), 0)
    o_ref[...] = jnp.where(rowid < n, h, 0.0)


def _l2_body(acc_ref, h1s_ref, degw_ref, g_ref, bt_ref, w_ref, b_ref, o_ref,
             *, n, npad):
    dinv = _dinv_from(degw_ref, npad)
    g = dinv * (acc_ref[0] + acc_ref[1] + h1s_ref[...])
    rowid = lax.broadcasted_iota(jnp.int32, (npad, 1), 0)
    rmask = (rowid < n).astype(jnp.float32)
    mean = jnp.sum(g, axis=0, keepdims=True) / n
    dev = (g - mean) * rmask
    var = jnp.sum(dev * dev, axis=0, keepdims=True) / n
    bn = g_ref[...] * (g - mean) * lax.rsqrt(var + 1e-5) + bt_ref[...]
    r = jnp.maximum(bn, 0.0)
    h2 = jnp.dot(r, w_ref[...], preferred_element_type=jnp.float32) + b_ref[...]
    o_ref[...] = jnp.where(rowid < n, dinv * h2, 0.0)


def _out_body(acc_ref, h2s_ref, degw_ref, o_ref, *, n, npad):
    dinv = _dinv_from(degw_ref, npad)
    o_ref[...] = (dinv * (acc_ref[0] + acc_ref[1] + h2s_ref[...]))[:n]


def _sc_deg_body(dstp_ref, zeros_ref, ones_ref, out_ref,
                 deg_sh, idx_v, ones_v, *, chpt, rpt):
    c = lax.axis_index("c")
    s = lax.axis_index("s")
    wid = s * NC + c
    rows = pl.ds(s * rpt, rpt)
    pltpu.sync_copy(zeros_ref.at[rows], deg_sh.at[rows])
    pltpu.sync_copy(ones_ref, ones_v)
    pltpu.sync_copy(dstp_ref.at[wid], idx_v)
    plsc.subcore_barrier()

    def body(j, carry):
        pltpu.sync_copy(ones_v, deg_sh.at[idx_v.at[j]], add=True)
        return carry

    lax.fori_loop(0, chpt, body, 0)
    plsc.subcore_barrier()
    pltpu.sync_copy(deg_sh.at[rows], out_ref.at[c].at[rows])


NBUF = 3


def _sc_spmm_body(srcp_ref, dstp_ref, hs_ref, zeros_ref, out_ref,
                  acc_sh, sring, dring, rows_v,
                  gs0, gs1, gs2, ds0, ds1, ds2, ss0, ss1, ss2,
                  *, chpt, rpt):
    c = lax.axis_index("c")
    s = lax.axis_index("s")
    wid = s * NC + c
    gsem = (gs0, gs1, gs2)
    dsem = (ds0, ds1, ds2)
    ssem = (ss0, ss1, ss2)
    rows = pl.ds(s * rpt, rpt)
    src_t = srcp_ref.at[wid]
    dst_t = dstp_ref.at[wid]
    pltpu.sync_copy(zeros_ref.at[rows], acc_sh.at[rows])
    for u in range(NBUF):
        pltpu.async_copy(src_t.at[u], sring.at[u], ssem[u])
        pltpu.async_copy(dst_t.at[u], dring.at[u], dsem[u])
    plsc.subcore_barrier()
    for u in range(2):
        pltpu.make_async_copy(dst_t.at[u], dring.at[u], dsem[u]).wait()
        pltpu.async_copy(hs_ref.at[dring.at[u]], rows_v.at[u], gsem[u])

    def body(i, carry):
        j0 = 3 * i
        for u in range(NBUF):
            j = j0 + u
            b = u
            b2 = (u + 2) % NBUF
            pltpu.make_async_copy(hs_ref.at[dring.at[b]], rows_v.at[b],
                                  gsem[b]).wait()

            @pl.when(j + 2 < chpt)
            def _(b2=b2, j=j):
                pltpu.make_async_copy(dst_t.at[j + 2], dring.at[b2],
                                      dsem[b2]).wait()
                pltpu.async_copy(hs_ref.at[dring.at[b2]], rows_v.at[b2],
                                 gsem[b2])

            pltpu.make_async_copy(src_t.at[j], sring.at[b], ssem[b]).wait()
            pltpu.sync_copy(rows_v.at[b], acc_sh.at[sring.at[b]], add=True)

            @pl.when(j + 3 < chpt)
            def _(b=b, j=j):
                pltpu.async_copy(src_t.at[j + 3], sring.at[b], ssem[b])
                pltpu.async_copy(dst_t.at[j + 3], dring.at[b], dsem[b])
        return carry

    lax.fori_loop(0, chpt // NBUF, body, 0)
    plsc.subcore_barrier()
    pltpu.sync_copy(acc_sh.at[rows], out_ref.at[c].at[rows])


def kernel(x, edge_index, W1, b1, gamma1, beta1, W2, b2):
    n, d = x.shape
    e = edge_index.shape[1]
    nchunks = e // CHK
    assert nchunks * CHK == e
    chpt = -(-nchunks // NW)
    chpt = ((chpt + NBUF - 1) // NBUF) * NBUF
    npad = ((n + 1 + 127) // 128) * 128
    rpt = npad // NS

    mesh = plsc.VectorSubcoreMesh(core_axis_name="c", subcore_axis_name="s")

    ei3 = edge_index.reshape(2, nchunks, CHK)
    srcp, dstp = pl.pallas_call(
        functools.partial(_edges_body, n=n, npad=npad, nchunks=nchunks),
        out_shape=[jax.ShapeDtypeStruct((NW * chpt, CHK), jnp.int32)] * 2,
    )(ei3)
    srcp = srcp.reshape(NW, chpt, CHK)
    dstp = dstp.reshape(NW, chpt, CHK)

    zeros16 = jnp.zeros((npad, 16), jnp.float32)
    zeros128 = jnp.zeros((npad, d), jnp.float32)
    ones16 = jnp.ones((CHK, 16), jnp.float32)
    x_pad = jnp.pad(x, ((0, npad - n), (0, 0)))

    deg_w = pl.kernel(
        functools.partial(_sc_deg_body, chpt=chpt, rpt=rpt),
        out_type=jax.ShapeDtypeStruct((NC, npad, 16), jnp.float32),
        mesh=mesh,
        compiler_params=pltpu.CompilerParams(use_tc_tiling_on_sc=False),
        scratch_types=[
            pltpu.VMEM_SHARED((npad, 16), jnp.float32),
            pltpu.VMEM((chpt, CHK), jnp.int32),
            pltpu.VMEM((CHK, 16), jnp.float32),
        ],
    )(dstp, zeros16, ones16)

    h1s = pl.pallas_call(
        functools.partial(_m1_body, n=n, npad=npad),
        out_shape=jax.ShapeDtypeStruct((npad, d), jnp.float32),
    )(x_pad, W1, b1.reshape(1, d), deg_w)

    spmm = pl.kernel(
        functools.partial(_sc_spmm_body, chpt=chpt, rpt=rpt),
        out_type=jax.ShapeDtypeStruct((NC, npad, d), jnp.float32),
        mesh=mesh,
        scratch_types=[
            pltpu.VMEM_SHARED((npad, d), jnp.float32),
            pltpu.VMEM((NBUF, CHK), jnp.int32),
            pltpu.VMEM((NBUF, CHK), jnp.int32),
            pltpu.VMEM((NBUF, CHK, d), jnp.float32),
        ] + [pltpu.SemaphoreType.DMA] * 9,
    )

    acc1 = spmm(srcp, dstp, h1s, zeros128)

    h2s = pl.pallas_call(
        functools.partial(_l2_body, n=n, npad=npad),
        out_shape=jax.ShapeDtypeStruct((npad, d), jnp.float32),
    )(acc1, h1s, deg_w, gamma1.reshape(1, d), beta1.reshape(1, d),
      W2, b2.reshape(1, d))

    acc2 = spmm(srcp, dstp, h2s, zeros128)

    out = pl.pallas_call(
        functools.partial(_out_body, n=n, npad=npad),
        out_shape=jax.ShapeDtypeStruct((n, d), jnp.float32),
    )(acc2, h2s, deg_w)
    return out


# speedup vs baseline: 31.1470x; 1.0118x over previous
import functools

import jax
import jax.numpy as jnp
from jax import lax
from jax.experimental import pallas as pl
from jax.experimental.pallas import tpu as pltpu
from jax.experimental.pallas import tpu_sc as plsc

NC = 2
NS = 16
NW = NC * NS
CHK = 128


def _edges_body(ei_ref, src_ref, dst_ref, *, n, npad, nchunks):
    src = ei_ref[0]
    dst = ei_ref[1]
    src_ref[:nchunks] = src
    dst_ref[:nchunks] = jnp.where(src == dst, n, dst)
    pad = src_ref.shape[0] - nchunks
    if pad:
        spare = npad - n - 1
        r = lax.broadcasted_iota(jnp.int32, (pad, CHK), 0)
        c = lax.broadcasted_iota(jnp.int32, (pad, CHK), 1)
        fill = n + 1 + lax.rem(r * CHK + c, jnp.int32(spare))
        src_ref[nchunks:] = fill
        dst_ref[nchunks:] = fill


def _dinv_from(degw, npad):
    deg = degw[0][:, :1] + degw[1][:, :1] + 1.0
    return lax.rsqrt(deg)


def _mm_body(x_ref, w_ref, b_ref, o_ref):
    o_ref[...] = jnp.dot(x_ref[...], w_ref[...],
                         preferred_element_type=jnp.float32) + b_ref[...]


def _scale_body(h_ref, degw_ref, o_ref, *, n, npad):
    dinv = _dinv_from(degw_ref, npad)
    rowid = lax.broadcasted_iota(jnp.int32, (npad, 1), 0)
    o_ref[...] = jnp.where(rowid < n, dinv * h_ref[...], 0.0)


def _l2_body(acc_ref, h1s_ref, degw_ref, g_ref, bt_ref, w_ref, b_ref, o_ref,
             *, n, npad):
    dinv = _dinv_from(degw_ref, npad)
    g = dinv * (acc_ref[0] + acc_ref[1] + h1s_ref[...])
    rowid = lax.broadcasted_iota(jnp.int32, (npad, 1), 0)
    rmask = (rowid < n).astype(jnp.float32)
    mean = jnp.sum(g, axis=0, keepdims=True) / n
    dev = (g - mean) * rmask
    var = jnp.sum(dev * dev, axis=0, keepdims=True) / n
    bn = g_ref[...] * (g - mean) * lax.rsqrt(var + 1e-5) + bt_ref[...]
    r = jnp.maximum(bn, 0.0)
    h2 = jnp.dot(r, w_ref[...], preferred_element_type=jnp.float32) + b_ref[...]
    o_ref[...] = jnp.where(rowid < n, dinv * h2, 0.0)


def _out_body(acc_ref, h2s_ref, degw_ref, o_ref, *, n, npad):
    dinv = _dinv_from(degw_ref, npad)
    o_ref[...] = (dinv * (acc_ref[0] + acc_ref[1] + h2s_ref[...]))[:n]


def _sc_deg_body(dstp_ref, zeros_ref, ones_ref, out_ref,
                 deg_sh, idx_v, ones_v, sem, *, chpt, rpt):
    c = lax.axis_index("c")
    s = lax.axis_index("s")
    wid = s * NC + c
    rows = pl.ds(s * rpt, rpt)
    pltpu.sync_copy(zeros_ref.at[rows], deg_sh.at[rows])
    pltpu.sync_copy(ones_ref, ones_v)
    pltpu.sync_copy(dstp_ref.at[wid], idx_v)
    plsc.subcore_barrier()

    def body(g, carry):
        for u in range(NBUF):
            pltpu.async_copy(ones_v, deg_sh.at[idx_v.at[NBUF * g + u]],
                             sem, add=True)
        for u in range(NBUF):
            pltpu.make_async_copy(ones_v, deg_sh.at[idx_v.at[0]], sem).wait()
        return carry

    lax.fori_loop(0, chpt // NBUF, body, 0)
    plsc.subcore_barrier()
    pltpu.sync_copy(deg_sh.at[rows], out_ref.at[c].at[rows])


NBUF = 3


def _sc_spmm_body(srcp_ref, dstp_ref, hs_ref, zeros_ref, out_ref,
                  acc_sh, sring, dring, rows_v,
                  gs0, gs1, gs2, ds0, ds1, ds2, ss0, ss1, ss2,
                  *, chpt, rpt):
    c = lax.axis_index("c")
    s = lax.axis_index("s")
    wid = s * NC + c
    gsem = (gs0, gs1, gs2)
    dsem = (ds0, ds1, ds2)
    ssem = (ss0, ss1, ss2)
    rows = pl.ds(s * rpt, rpt)
    src_t = srcp_ref.at[wid]
    dst_t = dstp_ref.at[wid]
    pltpu.sync_copy(zeros_ref.at[rows], acc_sh.at[rows])
    for u in range(NBUF):
        pltpu.async_copy(src_t.at[u], sring.at[u], ssem[u])
        pltpu.async_copy(dst_t.at[u], dring.at[u], dsem[u])
    plsc.subcore_barrier()
    for u in range(2):
        pltpu.make_async_copy(dst_t.at[u], dring.at[u], dsem[u]).wait()
        pltpu.async_copy(hs_ref.at[dring.at[u]], rows_v.at[u], gsem[u])

    def body(i, carry):
        j0 = 3 * i
        for u in range(NBUF):
            j = j0 + u
            b = u
            b2 = (u + 2) % NBUF
            pltpu.make_async_copy(hs_ref.at[dring.at[b]], rows_v.at[b],
                                  gsem[b]).wait()

            @pl.when(j + 2 < chpt)
            def _(b2=b2, j=j):
                pltpu.make_async_copy(dst_t.at[j + 2], dring.at[b2],
                                      dsem[b2]).wait()
                pltpu.async_copy(hs_ref.at[dring.at[b2]], rows_v.at[b2],
                                 gsem[b2])

            pltpu.make_async_copy(src_t.at[j], sring.at[b], ssem[b]).wait()
            pltpu.sync_copy(rows_v.at[b], acc_sh.at[sring.at[b]], add=True)

            @pl.when(j + 3 < chpt)
            def _(b=b, j=j):
                pltpu.async_copy(src_t.at[j + 3], sring.at[b], ssem[b])
                pltpu.async_copy(dst_t.at[j + 3], dring.at[b], dsem[b])
        return carry

    lax.fori_loop(0, chpt // NBUF, body, 0)
    plsc.subcore_barrier()
    pltpu.sync_copy(acc_sh.at[rows], out_ref.at[c].at[rows])


def kernel(x, edge_index, W1, b1, gamma1, beta1, W2, b2):
    n, d = x.shape
    e = edge_index.shape[1]
    nchunks = e // CHK
    assert nchunks * CHK == e
    chpt = -(-nchunks // NW)
    chpt = ((chpt + NBUF - 1) // NBUF) * NBUF
    npad = ((n + 1 + 127) // 128) * 128
    rpt = npad // NS

    mesh = plsc.VectorSubcoreMesh(core_axis_name="c", subcore_axis_name="s")

    ei3 = edge_index.reshape(2, nchunks, CHK)
    srcp, dstp = pl.pallas_call(
        functools.partial(_edges_body, n=n, npad=npad, nchunks=nchunks),
        out_shape=[jax.ShapeDtypeStruct((NW * chpt, CHK), jnp.int32)] * 2,
    )(ei3)
    srcp = srcp.reshape(NW, chpt, CHK)
    dstp = dstp.reshape(NW, chpt, CHK)

    zeros16 = jnp.zeros((npad, 16), jnp.float32)
    zeros128 = jnp.zeros((npad, d), jnp.float32)
    ones16 = jnp.ones((CHK, 16), jnp.float32)
    x_pad = jnp.pad(x, ((0, npad - n), (0, 0)))

    deg_w = pl.kernel(
        functools.partial(_sc_deg_body, chpt=chpt, rpt=rpt),
        out_type=jax.ShapeDtypeStruct((NC, npad, 16), jnp.float32),
        mesh=mesh,
        compiler_params=pltpu.CompilerParams(use_tc_tiling_on_sc=False),
        scratch_types=[
            pltpu.VMEM_SHARED((npad, 16), jnp.float32),
            pltpu.VMEM((chpt, CHK), jnp.int32),
            pltpu.VMEM((CHK, 16), jnp.float32),
            pltpu.SemaphoreType.DMA,
        ],
    )(dstp, zeros16, ones16)

    h1 = pl.pallas_call(
        _mm_body,
        out_shape=jax.ShapeDtypeStruct((npad, d), jnp.float32),
    )(x_pad, W1, b1.reshape(1, d))
    h1s = pl.pallas_call(
        functools.partial(_scale_body, n=n, npad=npad),
        out_shape=jax.ShapeDtypeStruct((npad, d), jnp.float32),
    )(h1, deg_w)

    spmm = pl.kernel(
        functools.partial(_sc_spmm_body, chpt=chpt, rpt=rpt),
        out_type=jax.ShapeDtypeStruct((NC, npad, d), jnp.float32),
        mesh=mesh,
        scratch_types=[
            pltpu.VMEM_SHARED((npad, d), jnp.float32),
            pltpu.VMEM((NBUF, CHK), jnp.int32),
            pltpu.VMEM((NBUF, CHK), jnp.int32),
            pltpu.VMEM((NBUF, CHK, d), jnp.float32),
        ] + [pltpu.SemaphoreType.DMA] * 9,
    )

    acc1 = spmm(srcp, dstp, h1s, zeros128)

    h2s = pl.pallas_call(
        functools.partial(_l2_body, n=n, npad=npad),
        out_shape=jax.ShapeDtypeStruct((npad, d), jnp.float32),
    )(acc1, h1s, deg_w, gamma1.reshape(1, d), beta1.reshape(1, d),
      W2, b2.reshape(1, d))

    acc2 = spmm(srcp, dstp, h2s, zeros128)

    out = pl.pallas_call(
        functools.partial(_out_body, n=n, npad=npad),
        out_shape=jax.ShapeDtypeStruct((n, d), jnp.float32),
    )(acc2, h2s, deg_w)
    return out
